# Initial kernel scaffold; baseline (speedup 1.0000x reference)
import jax
import jax.numpy as jnp
from jax.experimental import pallas as pl


def kernel(feature, keypoints):
    raise NotImplementedError("write your pallas kernel here")


# speedup vs baseline: 15.4027x; 15.4027x over previous
import functools

import jax
import jax.numpy as jnp
from jax import lax
from jax.experimental import pallas as pl
from jax.experimental.pallas import tpu as pltpu
from jax.experimental.pallas import tpu_sc as plsc

_IM_FE_RATIO = 4.0


@functools.lru_cache(maxsize=None)
def _build_sc_kernel(B, C, H, W, N):
    info = plsc.get_sparse_core_info()
    NC, NS, L = info.num_cores, info.num_subcores, info.num_lanes
    NW = NC * NS
    P = B * C
    assert P % NW == 0 and (P // NW) * NW == P
    planes_per_w = P // NW
    assert (planes_per_w * NW) // B * B == planes_per_w * NW
    workers_per_b = NW // B
    assert C % planes_per_w == 0
    HW = H * W
    n_chunks = N // L
    assert n_chunks * L == N

    mesh = plsc.VectorSubcoreMesh(core_axis_name="c", subcore_axis_name="s")

    @functools.partial(
        pl.kernel,
        out_type=jax.ShapeDtypeStruct((P, N), jnp.float32),
        mesh=mesh,
        compiler_params=pltpu.CompilerParams(needs_layout_passes=False),
        scratch_types=[
            pltpu.VMEM((N,), jnp.float32),
            pltpu.VMEM((N,), jnp.float32),
            pltpu.VMEM((4, N), jnp.int32),
            pltpu.VMEM((4, N), jnp.float32),
            pltpu.VMEM((HW,), jnp.float32),
            pltpu.VMEM((N,), jnp.float32),
        ],
    )
    def sc_kernel(feat_hbm, kpx_hbm, kpy_hbm, out_hbm,
                  kpx_v, kpy_v, idx_v, coef_v, plane_v, obuf_v):
        cid = lax.axis_index("c")
        sid = lax.axis_index("s")
        wid = sid * NC + cid
        b = wid // workers_per_b
        p0 = wid * planes_per_w

        pltpu.sync_copy(kpx_hbm.at[b], kpx_v)
        pltpu.sync_copy(kpy_hbm.at[b], kpy_v)

        inv_ratio = jnp.float32(1.0 / _IM_FE_RATIO)

        def precompute(i, carry):
            sl = pl.ds(i * L, L)
            x = kpx_v[sl]
            y = kpy_v[sl]
            mask = ((x > 1e-10).astype(jnp.float32)
                    + (y > 1e-10).astype(jnp.float32)) * jnp.float32(0.5)
            xs = x * inv_ratio
            ys = y * inv_ratio
            fxi = jnp.maximum(xs.astype(jnp.int32), 0)
            fyi = jnp.maximum(ys.astype(jnp.int32), 0)
            fxf = fxi.astype(jnp.float32)
            fyf = fyi.astype(jnp.float32)
            ux = xs - fxf
            lx = jnp.float32(1.0) - ux
            uy = ys - fyf
            ly = jnp.float32(1.0) - uy
            cxi = jnp.minimum(fxi + (xs > fxf).astype(jnp.int32), W - 1)
            cyi = jnp.minimum(fyi + (ys > fyf).astype(jnp.int32), H - 1)
            rf = fyi * W
            rc = cyi * W
            idx_v[0, sl] = rf + fxi
            idx_v[1, sl] = rf + cxi
            idx_v[2, sl] = rc + fxi
            idx_v[3, sl] = rc + cxi
            lym = ly * mask
            uym = uy * mask
            coef_v[0, sl] = lx * lym
            coef_v[1, sl] = ux * lym
            coef_v[2, sl] = lx * uym
            coef_v[3, sl] = ux * uym
            return carry

        lax.fori_loop(0, n_chunks, precompute, 0)

        def plane_loop(p, carry):
            pltpu.sync_copy(feat_hbm.at[p0 + p], plane_v)

            def chunk(i, c2):
                sl = pl.ds(i * L, L)
                a = plsc.load_gather(plane_v, [idx_v[0, sl]]) * coef_v[0, sl]
                a = a + plsc.load_gather(plane_v, [idx_v[1, sl]]) * coef_v[1, sl]
                a = a + plsc.load_gather(plane_v, [idx_v[2, sl]]) * coef_v[2, sl]
                a = a + plsc.load_gather(plane_v, [idx_v[3, sl]]) * coef_v[3, sl]
                obuf_v[sl] = a
                return c2

            lax.fori_loop(0, n_chunks, chunk, 0)
            pltpu.sync_copy(obuf_v, out_hbm.at[p0 + p])
            return carry

        lax.fori_loop(0, planes_per_w, plane_loop, 0)

    return sc_kernel


def kernel(feature, keypoints):
    B, C, H, W = feature.shape
    N = keypoints.shape[1]
    feat = feature.reshape(B * C, H * W)
    kpx = keypoints[:, :, 0]
    kpy = keypoints[:, :, 1]
    out = _build_sc_kernel(B, C, H, W, N)(feat, kpx, kpy)
    return out.reshape(B, C, N)


# speedup vs baseline: 28.7439x; 1.8662x over previous
import functools

import jax
import jax.numpy as jnp
from jax import lax
from jax.experimental import pallas as pl
from jax.experimental.pallas import tpu as pltpu
from jax.experimental.pallas import tpu_sc as plsc

_IM_FE_RATIO = 4.0


@functools.lru_cache(maxsize=None)
def _build_sc_kernel(B, C, H, W, N):
    info = plsc.get_sparse_core_info()
    NC, NS, L = info.num_cores, info.num_subcores, info.num_lanes
    NW = NC * NS
    P = B * C
    assert P % NW == 0
    planes_per_w = P // NW
    assert C % planes_per_w == 0
    assert N % L == 0

    mesh = plsc.VectorSubcoreMesh(core_axis_name="c", subcore_axis_name="s")

    @functools.partial(
        pl.kernel,
        out_type=jax.ShapeDtypeStruct((P, N), jnp.float32),
        mesh=mesh,
        compiler_params=pltpu.CompilerParams(needs_layout_passes=False),
        scratch_types=[
            pltpu.VMEM((N,), jnp.float32),
            pltpu.VMEM((N,), jnp.float32),
            pltpu.VMEM((4, N), jnp.int32),
            pltpu.VMEM((4, N), jnp.float32),
            pltpu.VMEM((H, W), jnp.float32),
            pltpu.VMEM((N,), jnp.float32),
        ],
    )
    def sc_kernel(feat_hbm, kpx_hbm, kpy_hbm, out_hbm,
                  kpx_v, kpy_v, idx_v, coef_v, plane_v, obuf_v):
        cid = lax.axis_index("c")
        sid = lax.axis_index("s")
        wid = sid * NC + cid
        p_lo = wid * planes_per_w
        b = p_lo // C
        c_lo = p_lo - b * C

        pltpu.sync_copy(kpx_hbm.at[b], kpx_v)
        pltpu.sync_copy(kpy_hbm.at[b], kpy_v)

        inv_ratio = jnp.float32(1.0 / _IM_FE_RATIO)

        @plsc.parallel_loop(0, N, step=L, unroll=2)
        def precompute(kb):
            x = kpx_v[pl.ds(kb, L)]
            y = kpy_v[pl.ds(kb, L)]
            mask = ((x > 1e-10).astype(jnp.float32)
                    + (y > 1e-10).astype(jnp.float32)) * jnp.float32(0.5)
            xs = x * inv_ratio
            ys = y * inv_ratio
            fxi = jnp.maximum(xs.astype(jnp.int32), 0)
            fyi = jnp.maximum(ys.astype(jnp.int32), 0)
            fxf = fxi.astype(jnp.float32)
            fyf = fyi.astype(jnp.float32)
            ux = xs - fxf
            lx = jnp.float32(1.0) - ux
            uy = ys - fyf
            ly = jnp.float32(1.0) - uy
            cxi = jnp.minimum(fxi + (xs > fxf).astype(jnp.int32), W - 1)
            cyi = jnp.minimum(fyi + (ys > fyf).astype(jnp.int32), H - 1)
            sl = pl.ds(kb, L)
            idx_v[0, sl] = fyi
            idx_v[1, sl] = cyi
            idx_v[2, sl] = fxi
            idx_v[3, sl] = cxi
            lym = ly * mask
            uym = uy * mask
            coef_v[0, sl] = lx * lym
            coef_v[1, sl] = ux * lym
            coef_v[2, sl] = lx * uym
            coef_v[3, sl] = ux * uym

        def plane_loop(p, carry):
            pltpu.sync_copy(feat_hbm.at[b, c_lo + p], plane_v)

            @plsc.parallel_loop(0, N, step=L, unroll=8)
            def chunk(kb):
                sl = pl.ds(kb, L)
                fy = idx_v[0, sl]
                cy = idx_v[1, sl]
                fx = idx_v[2, sl]
                cx = idx_v[3, sl]
                a = plsc.load_gather(plane_v, [fy, fx]) * coef_v[0, sl]
                a = a + plsc.load_gather(plane_v, [fy, cx]) * coef_v[1, sl]
                a = a + plsc.load_gather(plane_v, [cy, fx]) * coef_v[2, sl]
                a = a + plsc.load_gather(plane_v, [cy, cx]) * coef_v[3, sl]
                obuf_v[sl] = a

            pltpu.sync_copy(obuf_v, out_hbm.at[p_lo + p])
            return carry

        lax.fori_loop(0, planes_per_w, plane_loop, 0)

    return sc_kernel


def kernel(feature, keypoints):
    B, C, H, W = feature.shape
    N = keypoints.shape[1]
    kpx = keypoints[:, :, 0]
    kpy = keypoints[:, :, 1]
    out = _build_sc_kernel(B, C, H, W, N)(feature, kpx, kpy)
    return out.reshape(B, C, N)


# speedup vs baseline: 49.8757x; 1.7352x over previous
import functools

import jax
import jax.numpy as jnp
from jax import lax
from jax.experimental import pallas as pl
from jax.experimental.pallas import tpu as pltpu
from jax.experimental.pallas import tpu_sc as plsc

_IM_FE_RATIO = 4.0
_PP = 2


@functools.lru_cache(maxsize=None)
def _build_sc_kernel(B, C, H, W, N):
    info = plsc.get_sparse_core_info()
    NC, NS, L = info.num_cores, info.num_subcores, info.num_lanes
    NW = NC * NS
    P = B * C
    assert P % NW == 0
    planes_per_w = P // NW
    assert C % planes_per_w == 0
    assert N % L == 0
    n_groups = planes_per_w // _PP
    assert n_groups * _PP == planes_per_w and n_groups % 2 == 0

    mesh = plsc.VectorSubcoreMesh(core_axis_name="c", subcore_axis_name="s")

    @functools.partial(
        pl.kernel,
        out_type=jax.ShapeDtypeStruct((P, N), jnp.float32),
        mesh=mesh,
        compiler_params=pltpu.CompilerParams(needs_layout_passes=False),
        scratch_types=[
            pltpu.VMEM((N,), jnp.float32),
            pltpu.VMEM((N,), jnp.float32),
            pltpu.VMEM((4 * N,), jnp.int32),
            pltpu.VMEM((4 * N,), jnp.float32),
            pltpu.VMEM((_PP, H, W), jnp.float32),
            pltpu.VMEM((_PP, H, W), jnp.float32),
            pltpu.VMEM((_PP * N,), jnp.float32),
            pltpu.SemaphoreType.DMA,
            pltpu.SemaphoreType.DMA,
        ],
    )
    def sc_kernel(feat_hbm, kpx_hbm, kpy_hbm, out_hbm,
                  kpx_v, kpy_v, idx_v, coef_v, buf_a, buf_b, obuf_v,
                  sem_a, sem_b):
        cid = lax.axis_index("c")
        sid = lax.axis_index("s")
        wid = sid * NC + cid
        p_lo = wid * planes_per_w
        b = p_lo // C
        c_lo = p_lo - b * C

        pltpu.sync_copy(kpx_hbm.at[b], kpx_v)
        pltpu.sync_copy(kpy_hbm.at[b], kpy_v)

        inv_ratio = jnp.float32(1.0 / _IM_FE_RATIO)

        @plsc.parallel_loop(0, N, step=L, unroll=2)
        def precompute(kb):
            x = kpx_v[pl.ds(kb, L)]
            y = kpy_v[pl.ds(kb, L)]
            mask = ((x > 1e-10).astype(jnp.float32)
                    + (y > 1e-10).astype(jnp.float32)) * jnp.float32(0.5)
            xs = x * inv_ratio
            ys = y * inv_ratio
            fxi = jnp.maximum(xs.astype(jnp.int32), 0)
            fyi = jnp.maximum(ys.astype(jnp.int32), 0)
            fxf = fxi.astype(jnp.float32)
            fyf = fyi.astype(jnp.float32)
            ux = xs - fxf
            lx = jnp.float32(1.0) - ux
            uy = ys - fyf
            ly = jnp.float32(1.0) - uy
            cxi = jnp.minimum(fxi + (xs > fxf).astype(jnp.int32), W - 1)
            cyi = jnp.minimum(fyi + (ys > fyf).astype(jnp.int32), H - 1)
            idx_v[pl.ds(kb, L)] = fyi
            idx_v[pl.ds(N + kb, L)] = cyi
            idx_v[pl.ds(2 * N + kb, L)] = fxi
            idx_v[pl.ds(3 * N + kb, L)] = cxi
            lym = ly * mask
            uym = uy * mask
            coef_v[pl.ds(kb, L)] = lx * lym
            coef_v[pl.ds(N + kb, L)] = ux * lym
            coef_v[pl.ds(2 * N + kb, L)] = lx * uym
            coef_v[pl.ds(3 * N + kb, L)] = ux * uym

        def start_group(g, buf, sem):
            pltpu.async_copy(
                feat_hbm.at[b, pl.ds(c_lo + g * _PP, _PP)], buf, sem)

        def wait_group(buf, sem):
            pltpu.make_async_copy(
                feat_hbm.at[b, pl.ds(c_lo, _PP)], buf, sem).wait()

        def compute_group(g, buf):
            @plsc.parallel_loop(0, N, step=L, unroll=4)
            def chunk(kb):
                fy = idx_v[pl.ds(kb, L)]
                cy = idx_v[pl.ds(N + kb, L)]
                fx = idx_v[pl.ds(2 * N + kb, L)]
                cx = idx_v[pl.ds(3 * N + kb, L)]
                c0 = coef_v[pl.ds(kb, L)]
                c1 = coef_v[pl.ds(N + kb, L)]
                c2 = coef_v[pl.ds(2 * N + kb, L)]
                c3 = coef_v[pl.ds(3 * N + kb, L)]
                for j in range(_PP):
                    pj = buf.at[j]
                    a = plsc.load_gather(pj, [fy, fx]) * c0
                    a = a + plsc.load_gather(pj, [fy, cx]) * c1
                    a = a + plsc.load_gather(pj, [cy, fx]) * c2
                    a = a + plsc.load_gather(pj, [cy, cx]) * c3
                    obuf_v[pl.ds(j * N + kb, L)] = a

            for j in range(_PP):
                pltpu.sync_copy(obuf_v.at[pl.ds(j * N, N)],
                                out_hbm.at[p_lo + g * _PP + j])

        start_group(0, buf_a, sem_a)

        def pair_loop(i, carry):
            g0 = 2 * i
            wait_group(buf_a, sem_a)
            start_group(g0 + 1, buf_b, sem_b)
            compute_group(g0, buf_a)
            wait_group(buf_b, sem_b)

            @pl.when(i < n_groups // 2 - 1)
            def _():
                start_group(g0 + 2, buf_a, sem_a)

            compute_group(g0 + 1, buf_b)
            return carry

        lax.fori_loop(0, n_groups // 2, pair_loop, 0)

    return sc_kernel


def kernel(feature, keypoints):
    B, C, H, W = feature.shape
    N = keypoints.shape[1]
    kpx = keypoints[:, :, 0]
    kpy = keypoints[:, :, 1]
    out = _build_sc_kernel(B, C, H, W, N)(feature, kpx, kpy)
    return out.reshape(B, C, N)


# speedup vs baseline: 55.1396x; 1.1055x over previous
import functools

import jax
import jax.numpy as jnp
from jax import lax
from jax.experimental import pallas as pl
from jax.experimental.pallas import tpu as pltpu
from jax.experimental.pallas import tpu_sc as plsc

_IM_FE_RATIO = 4.0
_PP = 2


@functools.lru_cache(maxsize=None)
def _build_sc_kernel(B, C, H, W, N):
    info = plsc.get_sparse_core_info()
    NC, NS, L = info.num_cores, info.num_subcores, info.num_lanes
    NW = NC * NS
    P = B * C
    assert P % NW == 0
    planes_per_w = P // NW
    assert C % planes_per_w == 0
    assert N % L == 0
    n_groups = planes_per_w // _PP
    assert n_groups * _PP == planes_per_w and n_groups % 2 == 0

    mesh = plsc.VectorSubcoreMesh(core_axis_name="c", subcore_axis_name="s")

    @functools.partial(
        pl.kernel,
        out_type=jax.ShapeDtypeStruct((P, N), jnp.float32),
        mesh=mesh,
        compiler_params=pltpu.CompilerParams(needs_layout_passes=False),
        scratch_types=[
            pltpu.VMEM((N,), jnp.float32),
            pltpu.VMEM((N,), jnp.float32),
            pltpu.VMEM((N,), jnp.int32),
            pltpu.VMEM((4 * N,), jnp.float32),
            pltpu.VMEM((_PP, H, W), jnp.float32),
            pltpu.VMEM((_PP, H, W), jnp.float32),
            pltpu.VMEM((_PP * N,), jnp.float32),
            pltpu.SemaphoreType.DMA,
            pltpu.SemaphoreType.DMA,
        ],
    )
    def sc_kernel(feat_hbm, kpx_hbm, kpy_hbm, out_hbm,
                  kpx_v, kpy_v, idx_v, coef_v, buf_a, buf_b, obuf_v,
                  sem_a, sem_b):
        cid = lax.axis_index("c")
        sid = lax.axis_index("s")
        wid = sid * NC + cid
        p_lo = wid * planes_per_w
        b = p_lo // C
        c_lo = p_lo - b * C

        pltpu.sync_copy(kpx_hbm.at[b], kpx_v)
        pltpu.sync_copy(kpy_hbm.at[b], kpy_v)

        inv_ratio = jnp.float32(1.0 / _IM_FE_RATIO)

        @plsc.parallel_loop(0, N, step=L, unroll=2)
        def precompute(kb):
            x = kpx_v[pl.ds(kb, L)]
            y = kpy_v[pl.ds(kb, L)]
            mask = ((x > 1e-10).astype(jnp.float32)
                    + (y > 1e-10).astype(jnp.float32)) * jnp.float32(0.5)
            xs = x * inv_ratio
            ys = y * inv_ratio
            fxi = jnp.maximum(xs.astype(jnp.int32), 0)
            fyi = jnp.maximum(ys.astype(jnp.int32), 0)
            fxf = fxi.astype(jnp.float32)
            fyf = fyi.astype(jnp.float32)
            ux = xs - fxf
            lx = jnp.float32(1.0) - ux
            uy = ys - fyf
            ly = jnp.float32(1.0) - uy
            cxi = jnp.minimum(fxi + (xs > fxf).astype(jnp.int32), W - 1)
            cyi = jnp.minimum(fyi + (ys > fyf).astype(jnp.int32), H - 1)
            idx_v[pl.ds(kb, L)] = (fyi | (cyi << 7) | (fxi << 14)
                                   | (cxi << 21))
            lym = ly * mask
            uym = uy * mask
            coef_v[pl.ds(kb, L)] = lx * lym
            coef_v[pl.ds(N + kb, L)] = ux * lym
            coef_v[pl.ds(2 * N + kb, L)] = lx * uym
            coef_v[pl.ds(3 * N + kb, L)] = ux * uym

        def start_group(g, buf, sem):
            pltpu.async_copy(
                feat_hbm.at[b, pl.ds(c_lo + g * _PP, _PP)], buf, sem)

        def wait_group(buf, sem):
            pltpu.make_async_copy(
                feat_hbm.at[b, pl.ds(c_lo, _PP)], buf, sem).wait()

        def compute_group(g, buf):
            @plsc.parallel_loop(0, N, step=L, unroll=4)
            def chunk(kb):
                pc = idx_v[pl.ds(kb, L)]
                m7 = jnp.int32(127)
                fy = pc & m7
                cy = (pc >> 7) & m7
                fx = (pc >> 14) & m7
                cx = pc >> 21
                c0 = coef_v[pl.ds(kb, L)]
                c1 = coef_v[pl.ds(N + kb, L)]
                c2 = coef_v[pl.ds(2 * N + kb, L)]
                c3 = coef_v[pl.ds(3 * N + kb, L)]
                for j in range(_PP):
                    pj = buf.at[j]
                    a = plsc.load_gather(pj, [fy, fx]) * c0
                    a = a + plsc.load_gather(pj, [fy, cx]) * c1
                    a = a + plsc.load_gather(pj, [cy, fx]) * c2
                    a = a + plsc.load_gather(pj, [cy, cx]) * c3
                    obuf_v[pl.ds(j * N + kb, L)] = a

            for j in range(_PP):
                pltpu.sync_copy(obuf_v.at[pl.ds(j * N, N)],
                                out_hbm.at[p_lo + g * _PP + j])

        start_group(0, buf_a, sem_a)

        def pair_loop(i, carry):
            g0 = 2 * i
            wait_group(buf_a, sem_a)
            start_group(g0 + 1, buf_b, sem_b)
            compute_group(g0, buf_a)
            wait_group(buf_b, sem_b)

            @pl.when(i < n_groups // 2 - 1)
            def _():
                start_group(g0 + 2, buf_a, sem_a)

            compute_group(g0 + 1, buf_b)
            return carry

        lax.fori_loop(0, n_groups // 2, pair_loop, 0)

    return sc_kernel


def kernel(feature, keypoints):
    B, C, H, W = feature.shape
    N = keypoints.shape[1]
    kpx = keypoints[:, :, 0]
    kpy = keypoints[:, :, 1]
    out = _build_sc_kernel(B, C, H, W, N)(feature, kpx, kpy)
    return out.reshape(B, C, N)


# speedup vs baseline: 63.3817x; 1.1495x over previous
import functools

import jax
import jax.numpy as jnp
from jax import lax
from jax.experimental import pallas as pl
from jax.experimental.pallas import tpu as pltpu
from jax.experimental.pallas import tpu_sc as plsc

_IM_FE_RATIO = 4.0
_PP = 2


@functools.lru_cache(maxsize=None)
def _build_sc_kernel(B, C, H, W, N):
    info = plsc.get_sparse_core_info()
    NC, NS, L = info.num_cores, info.num_subcores, info.num_lanes
    NW = NC * NS
    P = B * C
    assert P % NW == 0
    planes_per_w = P // NW
    assert C % planes_per_w == 0
    assert N % L == 0
    n_groups = planes_per_w // _PP
    assert n_groups * _PP == planes_per_w and n_groups % 2 == 0

    mesh = plsc.VectorSubcoreMesh(core_axis_name="c", subcore_axis_name="s")

    @functools.partial(
        pl.kernel,
        out_type=jax.ShapeDtypeStruct((P, N), jnp.float32),
        mesh=mesh,
        compiler_params=pltpu.CompilerParams(needs_layout_passes=False),
        scratch_types=[
            pltpu.VMEM((N,), jnp.float32),
            pltpu.VMEM((N,), jnp.float32),
            pltpu.VMEM((N,), jnp.int32),
            pltpu.VMEM((N,), jnp.float32),
            pltpu.VMEM((N,), jnp.float32),
            pltpu.VMEM((_PP, H, W), jnp.float32),
            pltpu.VMEM((_PP, H, W), jnp.float32),
            pltpu.VMEM((_PP * N,), jnp.float32),
            pltpu.VMEM((_PP * N,), jnp.float32),
            pltpu.SemaphoreType.DMA,
            pltpu.SemaphoreType.DMA,
            pltpu.SemaphoreType.DMA,
            pltpu.SemaphoreType.DMA,
        ],
    )
    def sc_kernel(feat_hbm, kpx_hbm, kpy_hbm, out_hbm,
                  kpx_v, kpy_v, idx_v, ca_v, cu_v, buf_a, buf_b,
                  obuf_a, obuf_b, sem_a, sem_b, sem_oa, sem_ob):
        cid = lax.axis_index("c")
        sid = lax.axis_index("s")
        wid = sid * NC + cid
        p_lo = wid * planes_per_w
        b = p_lo // C
        c_lo = p_lo - b * C

        pltpu.sync_copy(kpx_hbm.at[b], kpx_v)
        pltpu.sync_copy(kpy_hbm.at[b], kpy_v)

        inv_ratio = jnp.float32(1.0 / _IM_FE_RATIO)

        @plsc.parallel_loop(0, N, step=L, unroll=2)
        def precompute(kb):
            x = kpx_v[pl.ds(kb, L)]
            y = kpy_v[pl.ds(kb, L)]
            code = ((x > 1e-10).astype(jnp.int32)
                    + (y > 1e-10).astype(jnp.int32))
            mask = code.astype(jnp.float32) * jnp.float32(0.5)
            xs = x * inv_ratio
            ys = y * inv_ratio
            fxi = jnp.maximum(xs.astype(jnp.int32), 0)
            fyi = jnp.maximum(ys.astype(jnp.int32), 0)
            fxf = fxi.astype(jnp.float32)
            fyf = fyi.astype(jnp.float32)
            ux = xs - fxf
            uy = ys - fyf
            cxi = jnp.minimum(fxi + (xs > fxf).astype(jnp.int32), W - 1)
            cyi = jnp.minimum(fyi + (ys > fyf).astype(jnp.int32), H - 1)
            idx_v[pl.ds(kb, L)] = (fyi | (cyi << 7) | (fxi << 14)
                                   | (cxi << 21) | (code << 28))
            ca_v[pl.ds(kb, L)] = ux * mask
            cu_v[pl.ds(kb, L)] = uy

        def start_group(g, buf, sem):
            pltpu.async_copy(
                feat_hbm.at[b, pl.ds(c_lo + g * _PP, _PP)], buf, sem)

        def wait_group(buf, sem):
            pltpu.make_async_copy(
                feat_hbm.at[b, pl.ds(c_lo, _PP)], buf, sem).wait()

        def start_out(g, obuf, sem):
            for j in range(_PP):
                pltpu.async_copy(obuf.at[pl.ds(j * N, N)],
                                 out_hbm.at[p_lo + g * _PP + j], sem)

        def wait_out(obuf, sem):
            for j in range(_PP):
                pltpu.make_async_copy(obuf.at[pl.ds(j * N, N)],
                                      out_hbm.at[p_lo + j], sem).wait()

        def compute_group(buf, obuf):
            @plsc.parallel_loop(0, N, step=L, unroll=4)
            def chunk(kb):
                pc = idx_v[pl.ds(kb, L)]
                m7 = jnp.int32(127)
                fy = pc & m7
                cy = (pc >> 7) & m7
                fx = (pc >> 14) & m7
                cx = (pc >> 21) & m7
                av = ca_v[pl.ds(kb, L)]
                uy = cu_v[pl.ds(kb, L)]
                m = (pc >> 28).astype(jnp.float32) * jnp.float32(0.5)
                lxm = m - av
                ly = jnp.float32(1.0) - uy
                for j in range(_PP):
                    pj = buf.at[j]
                    r0 = (plsc.load_gather(pj, [fy, fx]) * lxm
                          + plsc.load_gather(pj, [fy, cx]) * av)
                    r1 = (plsc.load_gather(pj, [cy, fx]) * lxm
                          + plsc.load_gather(pj, [cy, cx]) * av)
                    obuf[pl.ds(j * N + kb, L)] = r0 * ly + r1 * uy

        start_group(0, buf_a, sem_a)

        def pair_loop(i, carry):
            g0 = 2 * i
            wait_group(buf_a, sem_a)
            start_group(g0 + 1, buf_b, sem_b)

            @pl.when(i > 0)
            def _():
                wait_out(obuf_a, sem_oa)

            compute_group(buf_a, obuf_a)
            start_out(g0, obuf_a, sem_oa)

            wait_group(buf_b, sem_b)

            @pl.when(i < n_groups // 2 - 1)
            def _():
                start_group(g0 + 2, buf_a, sem_a)

            @pl.when(i > 0)
            def _():
                wait_out(obuf_b, sem_ob)

            compute_group(buf_b, obuf_b)
            start_out(g0 + 1, obuf_b, sem_ob)
            return carry

        lax.fori_loop(0, n_groups // 2, pair_loop, 0)
        wait_out(obuf_a, sem_oa)
        wait_out(obuf_b, sem_ob)

    return sc_kernel


def kernel(feature, keypoints):
    B, C, H, W = feature.shape
    N = keypoints.shape[1]
    kpx = keypoints[:, :, 0]
    kpy = keypoints[:, :, 1]
    out = _build_sc_kernel(B, C, H, W, N)(feature, kpx, kpy)
    return out.reshape(B, C, N)
